# baseline (speedup 1.0000x reference)
import functools

import jax
import jax.numpy as jnp
from jax.experimental import pallas as pl
from jax.experimental.pallas import tpu as pltpu

B = 2
N = 10000
E = 320000
HID = 128
OUT = 4 * HID
CIN = 256
BN = B * N
ETOT = E + N
AUGC = 640
RB = 1000
GRID = BN // RB


def _mm_body(x_ref, w_ref, o_ref):
    o_ref[...] = jnp.dot(x_ref[...], w_ref[...],
                         preferred_element_type=jnp.float32)


_mm = pl.pallas_call(
    _mm_body,
    grid=(GRID,),
    in_specs=[
        pl.BlockSpec((RB, CIN), lambda i: (i, 0)),
        pl.BlockSpec((CIN, AUGC), lambda i: (0, 0)),
    ],
    out_specs=pl.BlockSpec((RB, AUGC), lambda i: (i, 0)),
    out_shape=jax.ShapeDtypeStruct((BN, AUGC), jnp.float32),
)


def _lstm_body(conv_ref, c_ref, h_ref, cn_ref):
    conv = conv_ref[...]
    i = jax.nn.sigmoid(conv[:, 0 * HID:1 * HID])
    f = jax.nn.sigmoid(conv[:, 1 * HID:2 * HID])
    o = jax.nn.sigmoid(conv[:, 2 * HID:3 * HID])
    g = jnp.tanh(conv[:, 3 * HID:4 * HID])
    cn = f * c_ref[...] + i * g
    cn_ref[...] = cn
    h_ref[...] = o * jnp.tanh(cn)


_lstm = pl.pallas_call(
    _lstm_body,
    grid=(GRID,),
    in_specs=[
        pl.BlockSpec((RB, OUT), lambda i: (i, 0)),
        pl.BlockSpec((RB, HID), lambda i: (i, 0)),
    ],
    out_specs=[
        pl.BlockSpec((RB, HID), lambda i: (i, 0)),
        pl.BlockSpec((RB, HID), lambda i: (i, 0)),
    ],
    out_shape=[
        jax.ShapeDtypeStruct((BN, HID), jnp.float32),
        jax.ShapeDtypeStruct((BN, HID), jnp.float32),
    ],
)


def kernel(input_tensor, h_cur, c_cur, edge_index, W, att_src, att_dst, bias):
    combined = jnp.concatenate([input_tensor, h_cur], axis=2).reshape(BN, CIN)
    W_aug = jnp.zeros((CIN, AUGC), jnp.float32)
    W_aug = W_aug.at[:, :OUT].set(W)
    W_aug = W_aug.at[:, OUT].set(W @ att_src)
    W_aug = W_aug.at[:, OUT + 1].set(W @ att_dst)

    haug = _mm(combined, W_aug)
    h = haug[:, :OUT].reshape(B, N, OUT)
    a_src = haug[:, OUT].reshape(B, N)
    a_dst = haug[:, OUT + 1].reshape(B, N)

    loop = jnp.arange(N, dtype=edge_index.dtype)
    src = jnp.concatenate([edge_index[0], loop])
    dst = jnp.concatenate([edge_index[1], loop])

    def edge_phase(a_s, a_d, hb):
        alpha = jax.nn.leaky_relu(a_s[src] + a_d[dst], negative_slope=0.2)
        amax = jax.ops.segment_max(alpha, dst, num_segments=N)
        p = jnp.exp(alpha - amax[dst])
        denom = jax.ops.segment_sum(p, dst, num_segments=N)
        w = p / (denom[dst] + 1e-16)
        msg = hb[src] * w[:, None]
        return jax.ops.segment_sum(msg, dst, num_segments=N)

    conv = jax.vmap(edge_phase)(a_src, a_dst, h) + bias

    h_next, c_next = _lstm(conv.reshape(BN, OUT), c_cur.reshape(BN, HID))
    return (h_next.reshape(B, N, HID), c_next.reshape(B, N, HID))
